# Initial kernel scaffold; baseline (speedup 1.0000x reference)
import jax
import jax.numpy as jnp
from jax.experimental import pallas as pl


def kernel(p, fc_pos_W, fc_pos_b, W0, b0, W1, b1, Ws, fc_c_W, fc_c_b):
    raise NotImplementedError("write your pallas kernel here")


# speedup vs baseline: 2.9399x; 2.9399x over previous
import functools

import jax
import jax.numpy as jnp
from jax import lax
from jax.experimental import pallas as pl
from jax.experimental.pallas import tpu as pltpu
from jax.experimental.pallas import tpu_sc as plsc

B, T, DIM = 16, 4096, 3
HIDDEN = 32
H2 = 2 * HIDDEN
C_DIM = 32
RESO = 128
PAD = 0.1
NB = 5
S = RESO * RESO
L = 16
GROUPS = T // L
NEG = float(jnp.finfo(jnp.float32).min)


def _dot(a, b):
    return jax.lax.dot_general(a, b, (((1,), (0,)), ((), ())),
                               preferred_element_type=jnp.float32)


def _prologue_body(pt_ref, fw_ref, fb_ref, w0_ref, b0_ref, w1_ref, b1_ref,
                   ws_ref, idx_ref, net_ref):
    pt = pt_ref[0]
    x0 = pt[0:1, :]
    x1 = pt[2:3, :]

    def norm(v):
        vn = v / (1.0 + PAD + 10e-4)
        vn = vn + 0.5
        vn = jnp.where(vn >= 1.0, 1.0 - 10e-6, vn)
        vn = jnp.where(vn < 0.0, 0.0, vn)
        return vn

    xi0 = jnp.clip((norm(x0) * RESO).astype(jnp.int32), 0, RESO - 1)
    xi1 = jnp.clip((norm(x1) * RESO).astype(jnp.int32), 0, RESO - 1)
    idx_ref[0] = xi0 + RESO * xi1

    h = _dot(fw_ref[...], pt) + fb_ref[...]
    n0 = _dot(w0_ref[...], jnp.maximum(h, 0.0)) + b0_ref[...]
    dx = _dot(w1_ref[...], jnp.maximum(n0, 0.0)) + b1_ref[...]
    net_ref[0] = _dot(ws_ref[...], h) + dx


def _res_body(net_ref, pool_ref, w0_ref, b0_ref, w1_ref, b1_ref, ws_ref,
              out_ref):
    x = jnp.concatenate([net_ref[0], pool_ref[0]], axis=0)
    n0 = _dot(w0_ref[...], jnp.maximum(x, 0.0)) + b0_ref[...]
    dx = _dot(w1_ref[...], jnp.maximum(n0, 0.0)) + b1_ref[...]
    out_ref[0] = _dot(ws_ref[...], x) + dx


def _res_final_body(net_ref, pool_ref, w0_ref, b0_ref, w1_ref, b1_ref,
                    ws_ref, fcw_ref, fcb_ref, out_ref):
    x = jnp.concatenate([net_ref[0], pool_ref[0]], axis=0)
    n0 = _dot(w0_ref[...], jnp.maximum(x, 0.0)) + b0_ref[...]
    dx = _dot(w1_ref[...], jnp.maximum(n0, 0.0)) + b1_ref[...]
    net = _dot(ws_ref[...], x) + dx
    out_ref[0] = _dot(fcw_ref[...], net) + fcb_ref[...]


def _full(shape):
    return pl.BlockSpec(shape, lambda b: (0,) * len(shape))


def _row(shape):
    return pl.BlockSpec(shape, lambda b: (b,) + (0,) * (len(shape) - 1))


_prologue_call = pl.pallas_call(
    _prologue_body,
    grid=(B,),
    in_specs=[_row((1, DIM, T)), _full((H2, DIM)), _full((H2, 1)),
              _full((HIDDEN, H2)), _full((HIDDEN, 1)),
              _full((HIDDEN, HIDDEN)), _full((HIDDEN, 1)),
              _full((HIDDEN, H2))],
    out_specs=[_row((1, 1, T)), _row((1, HIDDEN, T))],
    out_shape=[jax.ShapeDtypeStruct((B, 1, T), jnp.int32),
               jax.ShapeDtypeStruct((B, HIDDEN, T), jnp.float32)],
)

_res_call = pl.pallas_call(
    _res_body,
    grid=(B,),
    in_specs=[_row((1, HIDDEN, T)), _row((1, HIDDEN, T)),
              _full((HIDDEN, H2)), _full((HIDDEN, 1)),
              _full((HIDDEN, HIDDEN)), _full((HIDDEN, 1)),
              _full((HIDDEN, H2))],
    out_specs=_row((1, HIDDEN, T)),
    out_shape=jax.ShapeDtypeStruct((B, HIDDEN, T), jnp.float32),
)

_res_final_call = pl.pallas_call(
    _res_final_body,
    grid=(B,),
    in_specs=[_row((1, HIDDEN, T)), _row((1, HIDDEN, T)),
              _full((HIDDEN, H2)), _full((HIDDEN, 1)),
              _full((HIDDEN, HIDDEN)), _full((HIDDEN, 1)),
              _full((HIDDEN, H2)), _full((C_DIM, HIDDEN)), _full((C_DIM, 1))],
    out_specs=_row((1, C_DIM, T)),
    out_shape=jax.ShapeDtypeStruct((B, C_DIM, T), jnp.float32),
)


_MESH = plsc.VectorSubcoreMesh(core_axis_name="c", subcore_axis_name="s")
_CH_PER_W = C_DIM // 2
_SC_PARAMS = pltpu.CompilerParams(needs_layout_passes=False)


@functools.partial(
    pl.kernel, mesh=_MESH,
    out_type=jax.ShapeDtypeStruct((B, C_DIM, T), jnp.float32),
    compiler_params=_SC_PARAMS,
    scratch_types=[pltpu.VMEM((T,), jnp.int32),
                   pltpu.VMEM((T,), jnp.float32),
                   pltpu.VMEM((S,), jnp.float32),
                   pltpu.VMEM((T,), jnp.float32)],
)
def _pool_call(idx_hbm, net_hbm, out_hbm, idx_v, val_v, tab, out_v):
    wid = lax.axis_index("c") * 16 + lax.axis_index("s")
    b = wid // 2
    c0 = (wid % 2) * _CH_PER_W
    pltpu.sync_copy(idx_hbm.at[b, 0], idx_v)

    def per_channel(ci, carry):
        pltpu.sync_copy(net_hbm.at[b, c0 + ci], val_v)

        def init_g(g, carry2):
            idxs = idx_v[pl.ds(g * L, L)]
            plsc.store_scatter(tab, [idxs], jnp.full((L,), NEG, jnp.float32))
            return carry2
        lax.fori_loop(0, GROUPS, init_g, 0)

        def rmw_g(g, carry2):
            idxs = idx_v[pl.ds(g * L, L)]
            vals = val_v[pl.ds(g * L, L)]
            cur = plsc.load_gather(tab, [idxs])
            act = vals > cur

            def cond(a):
                return jnp.any(a)

            def body(a):
                plsc.store_scatter(tab, [idxs], vals, mask=a)
                c2 = plsc.load_gather(tab, [idxs])
                return a & (vals > c2)

            lax.while_loop(cond, body, act)
            return carry2
        lax.fori_loop(0, GROUPS, rmw_g, 0)

        def gat_g(g, carry2):
            idxs = idx_v[pl.ds(g * L, L)]
            out_v[pl.ds(g * L, L)] = plsc.load_gather(tab, [idxs])
            return carry2
        lax.fori_loop(0, GROUPS, gat_g, 0)

        pltpu.sync_copy(out_v, out_hbm.at[b, c0 + ci])
        return carry
    lax.fori_loop(0, _CH_PER_W, per_channel, 0)


@functools.partial(
    pl.kernel, mesh=_MESH,
    out_type=jax.ShapeDtypeStruct((B, C_DIM, S), jnp.float32),
    compiler_params=_SC_PARAMS,
    scratch_types=[pltpu.VMEM((T,), jnp.int32),
                   pltpu.VMEM((T,), jnp.float32),
                   pltpu.VMEM((S,), jnp.int32),
                   pltpu.VMEM((S,), jnp.float32),
                   pltpu.VMEM((S,), jnp.float32)],
)
def _mean_call(idx_hbm, c_hbm, out_hbm, idx_v, val_v, claim, rec, tab):
    wid = lax.axis_index("c") * 16 + lax.axis_index("s")
    b = wid // 2
    c0 = (wid % 2) * _CH_PER_W
    pltpu.sync_copy(idx_hbm.at[b, 0], idx_v)
    lanes = lax.iota(jnp.int32, L)
    zeros = jnp.zeros((L,), jnp.float32)

    def zero_g(g, carry):
        tab[pl.ds(g * L, L)] = zeros
        return carry
    lax.fori_loop(0, S // L, zero_g, 0)

    def cnt_g(g, carry):
        idxs = idx_v[pl.ds(g * L, L)]

        def cond(a):
            return jnp.any(a)

        def body(a):
            plsc.store_scatter(claim, [idxs], lanes, mask=a)
            got = plsc.load_gather(claim, [idxs])
            win = a & (got == lanes)
            cur = plsc.load_gather(tab, [idxs])
            plsc.store_scatter(tab, [idxs], cur + 1.0, mask=win)
            return a & jnp.logical_not(win)

        lax.while_loop(cond, body, jnp.ones((L,), jnp.bool_))
        return carry
    lax.fori_loop(0, GROUPS, cnt_g, 0)

    def rec_g(g, carry):
        sl = pl.ds(g * L, L)
        rec[sl] = 1.0 / jnp.maximum(tab[sl], 1.0)
        return carry
    lax.fori_loop(0, S // L, rec_g, 0)

    def per_channel(ci, carry):
        pltpu.sync_copy(c_hbm.at[b, c0 + ci], val_v)
        lax.fori_loop(0, S // L, zero_g, 0)

        def add_g(g, carry2):
            idxs = idx_v[pl.ds(g * L, L)]
            vals = val_v[pl.ds(g * L, L)]
            vals = vals * plsc.load_gather(rec, [idxs])

            def cond(a):
                return jnp.any(a)

            def body(a):
                plsc.store_scatter(claim, [idxs], lanes, mask=a)
                got = plsc.load_gather(claim, [idxs])
                win = a & (got == lanes)
                cur = plsc.load_gather(tab, [idxs])
                plsc.store_scatter(tab, [idxs], cur + vals, mask=win)
                return a & jnp.logical_not(win)

            lax.while_loop(cond, body, jnp.ones((L,), jnp.bool_))
            return carry2
        lax.fori_loop(0, GROUPS, add_g, 0)

        pltpu.sync_copy(tab, out_hbm.at[b, c0 + ci])
        return carry
    lax.fori_loop(0, _CH_PER_W, per_channel, 0)


def kernel(p, fc_pos_W, fc_pos_b, W0, b0, W1, b1, Ws, fc_c_W, fc_c_b):
    pt = jnp.transpose(p, (0, 2, 1))
    fwT = jnp.transpose(fc_pos_W)
    fbT = fc_pos_b[:, None]
    w0T = jnp.transpose(W0, (0, 2, 1))
    b0T = b0[:, :, None]
    w1T = jnp.transpose(W1, (0, 2, 1))
    b1T = b1[:, :, None]
    wsT = jnp.transpose(Ws, (0, 2, 1))
    fcwT = jnp.transpose(fc_c_W)
    fcbT = fc_c_b[:, None]

    idx, net = _prologue_call(pt, fwT, fbT, w0T[0], b0T[0], w1T[0], b1T[0],
                              wsT[0])
    for i in range(1, NB):
        pooled = _pool_call(idx, net)
        if i < NB - 1:
            net = _res_call(net, pooled, w0T[i], b0T[i], w1T[i], b1T[i],
                            wsT[i])
        else:
            c = _res_final_call(net, pooled, w0T[i], b0T[i], w1T[i], b1T[i],
                                wsT[i], fcwT, fcbT)
    plane = _mean_call(idx, c)
    return plane.reshape(B, C_DIM, RESO, RESO)


# speedup vs baseline: 3.7118x; 1.2626x over previous
import functools

import jax
import jax.numpy as jnp
from jax import lax
from jax.experimental import pallas as pl
from jax.experimental.pallas import tpu as pltpu
from jax.experimental.pallas import tpu_sc as plsc

B, T, DIM = 16, 4096, 3
HIDDEN = 32
H2 = 2 * HIDDEN
C_DIM = 32
RESO = 128
PAD = 0.1
NB = 5
S = RESO * RESO
L = 16
GROUPS = T // L
NEG = float(jnp.finfo(jnp.float32).min)


def _dot(a, b):
    return jax.lax.dot_general(a, b, (((1,), (0,)), ((), ())),
                               preferred_element_type=jnp.float32)


def _prologue_body(pt_ref, fw_ref, fb_ref, w0_ref, b0_ref, w1_ref, b1_ref,
                   ws_ref, idx_ref, net_ref):
    pt = pt_ref[0]
    x0 = pt[0:1, :]
    x1 = pt[2:3, :]

    def norm(v):
        vn = v / (1.0 + PAD + 10e-4)
        vn = vn + 0.5
        vn = jnp.where(vn >= 1.0, 1.0 - 10e-6, vn)
        vn = jnp.where(vn < 0.0, 0.0, vn)
        return vn

    xi0 = jnp.clip((norm(x0) * RESO).astype(jnp.int32), 0, RESO - 1)
    xi1 = jnp.clip((norm(x1) * RESO).astype(jnp.int32), 0, RESO - 1)
    idx_ref[0] = xi0 + RESO * xi1

    h = _dot(fw_ref[...], pt) + fb_ref[...]
    n0 = _dot(w0_ref[...], jnp.maximum(h, 0.0)) + b0_ref[...]
    dx = _dot(w1_ref[...], jnp.maximum(n0, 0.0)) + b1_ref[...]
    net_ref[0] = _dot(ws_ref[...], h) + dx


def _res_body(net_ref, pool_ref, w0_ref, b0_ref, w1_ref, b1_ref, ws_ref,
              out_ref):
    x = jnp.concatenate([net_ref[0], pool_ref[0]], axis=0)
    n0 = _dot(w0_ref[...], jnp.maximum(x, 0.0)) + b0_ref[...]
    dx = _dot(w1_ref[...], jnp.maximum(n0, 0.0)) + b1_ref[...]
    out_ref[0] = _dot(ws_ref[...], x) + dx


def _res_final_body(net_ref, pool_ref, w0_ref, b0_ref, w1_ref, b1_ref,
                    ws_ref, fcw_ref, fcb_ref, out_ref):
    x = jnp.concatenate([net_ref[0], pool_ref[0]], axis=0)
    n0 = _dot(w0_ref[...], jnp.maximum(x, 0.0)) + b0_ref[...]
    dx = _dot(w1_ref[...], jnp.maximum(n0, 0.0)) + b1_ref[...]
    net = _dot(ws_ref[...], x) + dx
    out_ref[0] = _dot(fcw_ref[...], net) + fcb_ref[...]


def _full(shape):
    return pl.BlockSpec(shape, lambda b: (0,) * len(shape))


def _row(shape):
    return pl.BlockSpec(shape, lambda b: (b,) + (0,) * (len(shape) - 1))


_prologue_call = pl.pallas_call(
    _prologue_body,
    grid=(B,),
    in_specs=[_row((1, DIM, T)), _full((H2, DIM)), _full((H2, 1)),
              _full((HIDDEN, H2)), _full((HIDDEN, 1)),
              _full((HIDDEN, HIDDEN)), _full((HIDDEN, 1)),
              _full((HIDDEN, H2))],
    out_specs=[_row((1, 1, T)), _row((1, HIDDEN, T))],
    out_shape=[jax.ShapeDtypeStruct((B, 1, T), jnp.int32),
               jax.ShapeDtypeStruct((B, HIDDEN, T), jnp.float32)],
)

_res_call = pl.pallas_call(
    _res_body,
    grid=(B,),
    in_specs=[_row((1, HIDDEN, T)), _row((1, HIDDEN, T)),
              _full((HIDDEN, H2)), _full((HIDDEN, 1)),
              _full((HIDDEN, HIDDEN)), _full((HIDDEN, 1)),
              _full((HIDDEN, H2))],
    out_specs=_row((1, HIDDEN, T)),
    out_shape=jax.ShapeDtypeStruct((B, HIDDEN, T), jnp.float32),
)

_res_final_call = pl.pallas_call(
    _res_final_body,
    grid=(B,),
    in_specs=[_row((1, HIDDEN, T)), _row((1, HIDDEN, T)),
              _full((HIDDEN, H2)), _full((HIDDEN, 1)),
              _full((HIDDEN, HIDDEN)), _full((HIDDEN, 1)),
              _full((HIDDEN, H2)), _full((C_DIM, HIDDEN)), _full((C_DIM, 1))],
    out_specs=_row((1, C_DIM, T)),
    out_shape=jax.ShapeDtypeStruct((B, C_DIM, T), jnp.float32),
)


_MESH = plsc.VectorSubcoreMesh(core_axis_name="c", subcore_axis_name="s")
_CH_PER_W = C_DIM // 2
_SC_PARAMS = pltpu.CompilerParams(needs_layout_passes=False)


def _build_schedule(idx_v, claim, rep_v, nf_pt_v):
    lanes = lax.iota(jnp.int32, L)

    @plsc.parallel_loop(0, GROUPS, unroll=4)
    def _(g):
        idxs = idx_v[pl.ds(g * L, L)]
        plsc.store_scatter(claim, [idxs], jnp.full((L,), -1, jnp.int32))

    def build_g(g, off):
        idxs = idx_v[pl.ds(g * L, L)]
        gids = g * L + lanes
        cur = plsc.load_gather(claim, [idxs])
        free = cur == -1
        plsc.store_scatter(claim, [idxs], gids, mask=free)
        got = plsc.load_gather(claim, [idxs])
        rep = free & (got == gids)
        rep_v[pl.ds(g * L, L)] = jnp.where(rep, 1, 0)
        nf = jnp.logical_not(rep)
        nf_i = jnp.where(nf, 1, 0)
        pos = off + plsc.cumsum(nf_i) - 1
        plsc.store_scatter(nf_pt_v, [pos], gids, mask=nf)
        return off + jnp.sum(nf_i)

    return lax.fori_loop(0, GROUPS, build_g, jnp.int32(0))


@functools.partial(
    pl.kernel, mesh=_MESH,
    out_type=jax.ShapeDtypeStruct((B, C_DIM, T), jnp.float32),
    compiler_params=_SC_PARAMS,
    scratch_types=[pltpu.VMEM((T,), jnp.int32),
                   pltpu.VMEM((T,), jnp.float32),
                   pltpu.VMEM((S,), jnp.float32),
                   pltpu.VMEM((T,), jnp.float32),
                   pltpu.VMEM((S,), jnp.int32),
                   pltpu.VMEM((T,), jnp.int32),
                   pltpu.VMEM((T,), jnp.int32)],
)
def _pool_call(idx_hbm, net_hbm, out_hbm, idx_v, val_v, tab, out_v, claim,
               rep_v, nf_pt_v):
    wid = lax.axis_index("c") * 16 + lax.axis_index("s")
    b = wid // 2
    c0 = (wid % 2) * _CH_PER_W
    pltpu.sync_copy(idx_hbm.at[b, 0], idx_v)
    n_nf = _build_schedule(idx_v, claim, rep_v, nf_pt_v)
    n_nf_vregs = (n_nf + L - 1) // L
    lanes = lax.iota(jnp.int32, L)

    def per_channel(ci, carry):
        pltpu.sync_copy(net_hbm.at[b, c0 + ci], val_v)

        @plsc.parallel_loop(0, GROUPS, unroll=4)
        def _(g):
            sl = pl.ds(g * L, L)
            rep = rep_v[sl] != 0
            plsc.store_scatter(tab, [idx_v[sl]], val_v[sl], mask=rep)

        def nf_k(k, carry2):
            valid = (k * L + lanes) < n_nf
            pts = nf_pt_v[pl.ds(k * L, L)]
            pts = jnp.where(valid, pts, 0)
            cells = plsc.load_gather(idx_v, [pts])
            vals = plsc.load_gather(val_v, [pts])

            def cond(a):
                return jnp.any(a)

            def body(a):
                cur = plsc.load_gather(tab, [cells])
                need = a & (vals > cur)
                plsc.store_scatter(tab, [cells], vals, mask=need)
                got = plsc.load_gather(tab, [cells])
                return a & (vals > got)

            lax.while_loop(cond, body, valid)
            return carry2
        lax.fori_loop(0, n_nf_vregs, nf_k, 0)

        @plsc.parallel_loop(0, GROUPS, unroll=4)
        def _(g):
            sl = pl.ds(g * L, L)
            out_v[sl] = plsc.load_gather(tab, [idx_v[sl]])

        pltpu.sync_copy(out_v, out_hbm.at[b, c0 + ci])
        return carry
    lax.fori_loop(0, _CH_PER_W, per_channel, 0)


@functools.partial(
    pl.kernel, mesh=_MESH,
    out_type=jax.ShapeDtypeStruct((B, C_DIM, S), jnp.float32),
    compiler_params=_SC_PARAMS,
    scratch_types=[pltpu.VMEM((T,), jnp.int32),
                   pltpu.VMEM((T,), jnp.float32),
                   pltpu.VMEM((S,), jnp.int32),
                   pltpu.VMEM((S,), jnp.float32),
                   pltpu.VMEM((S,), jnp.float32),
                   pltpu.VMEM((T,), jnp.int32),
                   pltpu.VMEM((T,), jnp.int32)],
)
def _mean_call(idx_hbm, c_hbm, out_hbm, idx_v, val_v, claim, rec, tab,
               rep_v, nf_pt_v):
    wid = lax.axis_index("c") * 16 + lax.axis_index("s")
    b = wid // 2
    c0 = (wid % 2) * _CH_PER_W
    pltpu.sync_copy(idx_hbm.at[b, 0], idx_v)
    n_nf = _build_schedule(idx_v, claim, rep_v, nf_pt_v)
    n_nf_vregs = (n_nf + L - 1) // L
    lanes = lax.iota(jnp.int32, L)

    @plsc.parallel_loop(0, GROUPS, unroll=4)
    def _(g):
        sl = pl.ds(g * L, L)
        rep = rep_v[sl] != 0
        plsc.store_scatter(rec, [idx_v[sl]], jnp.ones((L,), jnp.float32),
                           mask=rep)

    def cnt_k(k, carry):
        valid = (k * L + lanes) < n_nf
        pts = nf_pt_v[pl.ds(k * L, L)]
        pts = jnp.where(valid, pts, 0)
        cells = plsc.load_gather(idx_v, [pts])

        def cond(a):
            return jnp.any(a)

        def body(a):
            plsc.store_scatter(claim, [cells], lanes, mask=a)
            got = plsc.load_gather(claim, [cells])
            win = a & (got == lanes)
            cur = plsc.load_gather(rec, [cells])
            plsc.store_scatter(rec, [cells], cur + 1.0, mask=win)
            return a & jnp.logical_not(win)

        lax.while_loop(cond, body, valid)
        return carry
    lax.fori_loop(0, n_nf_vregs, cnt_k, 0)

    @plsc.parallel_loop(0, S // L, unroll=4)
    def _(g):
        sl = pl.ds(g * L, L)
        rec[sl] = 1.0 / jnp.maximum(rec[sl], 1.0)

    @plsc.parallel_loop(0, S // L, unroll=4)
    def _(g):
        tab[pl.ds(g * L, L)] = jnp.zeros((L,), jnp.float32)

    def per_channel(ci, carry):
        pltpu.sync_copy(c_hbm.at[b, c0 + ci], val_v)

        @plsc.parallel_loop(0, GROUPS, unroll=4)
        def _(g):
            sl = pl.ds(g * L, L)
            rep = rep_v[sl] != 0
            cells = idx_v[sl]
            sval = val_v[sl] * plsc.load_gather(rec, [cells])
            plsc.store_scatter(tab, [cells], sval, mask=rep)

        def add_k(k, carry2):
            valid = (k * L + lanes) < n_nf
            pts = nf_pt_v[pl.ds(k * L, L)]
            pts = jnp.where(valid, pts, 0)
            cells = plsc.load_gather(idx_v, [pts])
            vals = plsc.load_gather(val_v, [pts])
            sval = vals * plsc.load_gather(rec, [cells])

            def cond(a):
                return jnp.any(a)

            def body(a):
                plsc.store_scatter(claim, [cells], lanes, mask=a)
                got = plsc.load_gather(claim, [cells])
                win = a & (got == lanes)
                cur = plsc.load_gather(tab, [cells])
                plsc.store_scatter(tab, [cells], cur + sval, mask=win)
                return a & jnp.logical_not(win)

            lax.while_loop(cond, body, valid)
            return carry2
        lax.fori_loop(0, n_nf_vregs, add_k, 0)

        pltpu.sync_copy(tab, out_hbm.at[b, c0 + ci])
        return carry
    lax.fori_loop(0, _CH_PER_W, per_channel, 0)


def kernel(p, fc_pos_W, fc_pos_b, W0, b0, W1, b1, Ws, fc_c_W, fc_c_b):
    pt = jnp.transpose(p, (0, 2, 1))
    fwT = jnp.transpose(fc_pos_W)
    fbT = fc_pos_b[:, None]
    w0T = jnp.transpose(W0, (0, 2, 1))
    b0T = b0[:, :, None]
    w1T = jnp.transpose(W1, (0, 2, 1))
    b1T = b1[:, :, None]
    wsT = jnp.transpose(Ws, (0, 2, 1))
    fcwT = jnp.transpose(fc_c_W)
    fcbT = fc_c_b[:, None]

    idx, net = _prologue_call(pt, fwT, fbT, w0T[0], b0T[0], w1T[0], b1T[0],
                              wsT[0])
    for i in range(1, NB):
        pooled = _pool_call(idx, net)
        if i < NB - 1:
            net = _res_call(net, pooled, w0T[i], b0T[i], w1T[i], b1T[i],
                            wsT[i])
        else:
            c = _res_final_call(net, pooled, w0T[i], b0T[i], w1T[i], b1T[i],
                                wsT[i], fcwT, fcbT)
    plane = _mean_call(idx, c)
    return plane.reshape(B, C_DIM, RESO, RESO)


# speedup vs baseline: 3.8938x; 1.0490x over previous
import functools

import jax
import jax.numpy as jnp
from jax import lax
from jax.experimental import pallas as pl
from jax.experimental.pallas import tpu as pltpu
from jax.experimental.pallas import tpu_sc as plsc

B, T, DIM = 16, 4096, 3
HIDDEN = 32
H2 = 2 * HIDDEN
C_DIM = 32
RESO = 128
PAD = 0.1
NB = 5
S = RESO * RESO
L = 16
GROUPS = T // L
NEG = float(jnp.finfo(jnp.float32).min)


def _dot(a, b):
    return jax.lax.dot_general(a, b, (((1,), (0,)), ((), ())),
                               preferred_element_type=jnp.float32)


def _prologue_body(pt_ref, fw_ref, fb_ref, w0_ref, b0_ref, w1_ref, b1_ref,
                   ws_ref, idx_ref, net_ref):
    pt = pt_ref[0]
    x0 = pt[0:1, :]
    x1 = pt[2:3, :]

    def norm(v):
        vn = v / (1.0 + PAD + 10e-4)
        vn = vn + 0.5
        vn = jnp.where(vn >= 1.0, 1.0 - 10e-6, vn)
        vn = jnp.where(vn < 0.0, 0.0, vn)
        return vn

    xi0 = jnp.clip((norm(x0) * RESO).astype(jnp.int32), 0, RESO - 1)
    xi1 = jnp.clip((norm(x1) * RESO).astype(jnp.int32), 0, RESO - 1)
    idx_ref[0] = xi0 + RESO * xi1

    h = _dot(fw_ref[...], pt) + fb_ref[...]
    n0 = _dot(w0_ref[...], jnp.maximum(h, 0.0)) + b0_ref[...]
    dx = _dot(w1_ref[...], jnp.maximum(n0, 0.0)) + b1_ref[...]
    net_ref[0] = _dot(ws_ref[...], h) + dx


def _res_body(net_ref, pool_ref, w0_ref, b0_ref, w1_ref, b1_ref, ws_ref,
              out_ref):
    x = jnp.concatenate([net_ref[0], pool_ref[0]], axis=0)
    n0 = _dot(w0_ref[...], jnp.maximum(x, 0.0)) + b0_ref[...]
    dx = _dot(w1_ref[...], jnp.maximum(n0, 0.0)) + b1_ref[...]
    out_ref[0] = _dot(ws_ref[...], x) + dx


def _res_final_body(net_ref, pool_ref, w0_ref, b0_ref, w1_ref, b1_ref,
                    ws_ref, fcw_ref, fcb_ref, out_ref):
    x = jnp.concatenate([net_ref[0], pool_ref[0]], axis=0)
    n0 = _dot(w0_ref[...], jnp.maximum(x, 0.0)) + b0_ref[...]
    dx = _dot(w1_ref[...], jnp.maximum(n0, 0.0)) + b1_ref[...]
    net = _dot(ws_ref[...], x) + dx
    out_ref[0] = _dot(fcw_ref[...], net) + fcb_ref[...]


def _full(shape):
    return pl.BlockSpec(shape, lambda b: (0,) * len(shape))


def _row(shape):
    return pl.BlockSpec(shape, lambda b: (b,) + (0,) * (len(shape) - 1))


_prologue_call = pl.pallas_call(
    _prologue_body,
    grid=(B,),
    in_specs=[_row((1, DIM, T)), _full((H2, DIM)), _full((H2, 1)),
              _full((HIDDEN, H2)), _full((HIDDEN, 1)),
              _full((HIDDEN, HIDDEN)), _full((HIDDEN, 1)),
              _full((HIDDEN, H2))],
    out_specs=[_row((1, 1, T)), _row((1, HIDDEN, T))],
    out_shape=[jax.ShapeDtypeStruct((B, 1, T), jnp.int32),
               jax.ShapeDtypeStruct((B, HIDDEN, T), jnp.float32)],
)

_res_call = pl.pallas_call(
    _res_body,
    grid=(B,),
    in_specs=[_row((1, HIDDEN, T)), _row((1, HIDDEN, T)),
              _full((HIDDEN, H2)), _full((HIDDEN, 1)),
              _full((HIDDEN, HIDDEN)), _full((HIDDEN, 1)),
              _full((HIDDEN, H2))],
    out_specs=_row((1, HIDDEN, T)),
    out_shape=jax.ShapeDtypeStruct((B, HIDDEN, T), jnp.float32),
)

_res_final_call = pl.pallas_call(
    _res_final_body,
    grid=(B,),
    in_specs=[_row((1, HIDDEN, T)), _row((1, HIDDEN, T)),
              _full((HIDDEN, H2)), _full((HIDDEN, 1)),
              _full((HIDDEN, HIDDEN)), _full((HIDDEN, 1)),
              _full((HIDDEN, H2)), _full((C_DIM, HIDDEN)), _full((C_DIM, 1))],
    out_specs=_row((1, C_DIM, T)),
    out_shape=jax.ShapeDtypeStruct((B, C_DIM, T), jnp.float32),
)


_MESH = plsc.VectorSubcoreMesh(core_axis_name="c", subcore_axis_name="s")
_CH_PER_W = C_DIM // 2
_SC_PARAMS = pltpu.CompilerParams(needs_layout_passes=False)


def _build_schedule(idx_v, claim, rep_v, nf_pt_v):
    lanes = lax.iota(jnp.int32, L)

    @plsc.parallel_loop(0, GROUPS, unroll=4)
    def _(g):
        idxs = idx_v[pl.ds(g * L, L)]
        plsc.store_scatter(claim, [idxs], jnp.full((L,), -1, jnp.int32))

    def build_g(g, off):
        idxs = idx_v[pl.ds(g * L, L)]
        gids = g * L + lanes
        cur = plsc.load_gather(claim, [idxs])
        free = cur == -1
        plsc.store_scatter(claim, [idxs], gids, mask=free)
        got = plsc.load_gather(claim, [idxs])
        rep = free & (got == gids)
        rep_v[pl.ds(g * L, L)] = jnp.where(rep, 1, 0)
        nf = jnp.logical_not(rep)
        nf_i = jnp.where(nf, 1, 0)
        pos = off + plsc.cumsum(nf_i) - 1
        plsc.store_scatter(nf_pt_v, [pos], gids, mask=nf)
        return off + jnp.sum(nf_i)

    return lax.fori_loop(0, GROUPS, build_g, jnp.int32(0))


@functools.partial(
    pl.kernel, mesh=_MESH,
    out_type=jax.ShapeDtypeStruct((B, C_DIM, T), jnp.float32),
    compiler_params=_SC_PARAMS,
    scratch_types=[pltpu.VMEM((T,), jnp.int32),
                   pltpu.VMEM((T,), jnp.float32),
                   pltpu.VMEM((T,), jnp.float32),
                   pltpu.VMEM((S,), jnp.float32),
                   pltpu.VMEM((T,), jnp.float32),
                   pltpu.VMEM((T,), jnp.float32),
                   pltpu.VMEM((S,), jnp.int32),
                   pltpu.VMEM((T,), jnp.int32),
                   pltpu.VMEM((T,), jnp.int32),
                   pltpu.SemaphoreType.DMA,
                   pltpu.SemaphoreType.DMA,
                   pltpu.SemaphoreType.DMA,
                   pltpu.SemaphoreType.DMA],
)
def _pool_call(idx_hbm, net_hbm, out_hbm, idx_v, in_a, in_b, tab, out_a,
               out_b, claim, rep_v, nf_pt_v, sem_ia, sem_ib, sem_oa, sem_ob):
    wid = lax.axis_index("c") * 16 + lax.axis_index("s")
    b = wid // 2
    c0 = (wid % 2) * _CH_PER_W
    pltpu.sync_copy(idx_hbm.at[b, 0], idx_v)
    in_pend = [pltpu.async_copy(net_hbm.at[b, c0], in_a, sem_ia),
               pltpu.async_copy(net_hbm.at[b, c0 + 1], in_b, sem_ib)]
    n_nf = _build_schedule(idx_v, claim, rep_v, nf_pt_v)
    n_nf_vregs = (n_nf + L - 1) // L
    lanes = lax.iota(jnp.int32, L)

    pend = [None, None]
    for ci in range(_CH_PER_W):
        p = ci % 2
        vv, ob = (in_a, out_a) if p == 0 else (in_b, out_b)
        sem_i, sem_o = (sem_ia, sem_oa) if p == 0 else (sem_ib, sem_ob)
        in_pend[p].wait()
        if pend[p] is not None:
            pend[p].wait()

        @plsc.parallel_loop(0, GROUPS, unroll=4)
        def _(g, vv=vv):
            sl = pl.ds(g * L, L)
            rep = rep_v[sl] != 0
            plsc.store_scatter(tab, [idx_v[sl]], vv[sl], mask=rep)

        def nf_k(k, carry2, vv=vv):
            valid = (k * L + lanes) < n_nf
            pts = nf_pt_v[pl.ds(k * L, L)]
            pts = jnp.where(valid, pts, 0)
            cells = plsc.load_gather(idx_v, [pts])
            vals = plsc.load_gather(vv, [pts])

            def cond(a):
                return jnp.any(a)

            def body(a):
                cur = plsc.load_gather(tab, [cells])
                need = a & (vals > cur)
                plsc.store_scatter(tab, [cells], vals, mask=need)
                got = plsc.load_gather(tab, [cells])
                return a & (vals > got)

            lax.while_loop(cond, body, valid)
            return carry2
        lax.fori_loop(0, n_nf_vregs, nf_k, 0)

        @plsc.parallel_loop(0, GROUPS, unroll=4)
        def _(g, ob=ob):
            sl = pl.ds(g * L, L)
            ob[sl] = plsc.load_gather(tab, [idx_v[sl]])

        if ci + 2 < _CH_PER_W:
            in_pend[p] = pltpu.async_copy(net_hbm.at[b, c0 + ci + 2], vv,
                                          sem_i)
        pend[p] = pltpu.async_copy(ob, out_hbm.at[b, c0 + ci], sem_o)
    pend[0].wait()
    pend[1].wait()


@functools.partial(
    pl.kernel, mesh=_MESH,
    out_type=jax.ShapeDtypeStruct((B, C_DIM, S), jnp.float32),
    compiler_params=_SC_PARAMS,
    scratch_types=[pltpu.VMEM((T,), jnp.int32),
                   pltpu.VMEM((T,), jnp.float32),
                   pltpu.VMEM((T,), jnp.float32),
                   pltpu.VMEM((S,), jnp.int32),
                   pltpu.VMEM((S,), jnp.float32),
                   pltpu.VMEM((S,), jnp.float32),
                   pltpu.VMEM((S,), jnp.float32),
                   pltpu.VMEM((T,), jnp.int32),
                   pltpu.VMEM((T,), jnp.int32),
                   pltpu.SemaphoreType.DMA,
                   pltpu.SemaphoreType.DMA,
                   pltpu.SemaphoreType.DMA,
                   pltpu.SemaphoreType.DMA],
)
def _mean_call(idx_hbm, c_hbm, out_hbm, idx_v, in_a, in_b, claim, rec,
               tab_a, tab_b, rep_v, nf_pt_v, sem_ia, sem_ib, sem_oa, sem_ob):
    wid = lax.axis_index("c") * 16 + lax.axis_index("s")
    b = wid // 2
    c0 = (wid % 2) * _CH_PER_W
    pltpu.sync_copy(idx_hbm.at[b, 0], idx_v)
    in_pend = [pltpu.async_copy(c_hbm.at[b, c0], in_a, sem_ia),
               pltpu.async_copy(c_hbm.at[b, c0 + 1], in_b, sem_ib)]
    n_nf = _build_schedule(idx_v, claim, rep_v, nf_pt_v)
    n_nf_vregs = (n_nf + L - 1) // L
    lanes = lax.iota(jnp.int32, L)

    @plsc.parallel_loop(0, GROUPS, unroll=4)
    def _(g):
        sl = pl.ds(g * L, L)
        rep = rep_v[sl] != 0
        plsc.store_scatter(rec, [idx_v[sl]], jnp.ones((L,), jnp.float32),
                           mask=rep)

    def cnt_k(k, carry):
        valid = (k * L + lanes) < n_nf
        pts = nf_pt_v[pl.ds(k * L, L)]
        pts = jnp.where(valid, pts, 0)
        cells = plsc.load_gather(idx_v, [pts])

        def cond(a):
            return jnp.any(a)

        def body(a):
            plsc.store_scatter(claim, [cells], lanes, mask=a)
            got = plsc.load_gather(claim, [cells])
            win = a & (got == lanes)
            cur = plsc.load_gather(rec, [cells])
            plsc.store_scatter(rec, [cells], cur + 1.0, mask=win)
            return a & jnp.logical_not(win)

        lax.while_loop(cond, body, valid)
        return carry
    lax.fori_loop(0, n_nf_vregs, cnt_k, 0)

    @plsc.parallel_loop(0, S // L, unroll=4)
    def _(g):
        sl = pl.ds(g * L, L)
        rec[sl] = 1.0 / jnp.maximum(rec[sl], 1.0)

    @plsc.parallel_loop(0, S // L, unroll=4)
    def _(g):
        tab_a[pl.ds(g * L, L)] = jnp.zeros((L,), jnp.float32)

    @plsc.parallel_loop(0, S // L, unroll=4)
    def _(g):
        tab_b[pl.ds(g * L, L)] = jnp.zeros((L,), jnp.float32)

    out_pend = [None, None]
    for ci in range(_CH_PER_W):
        p = ci % 2
        vv, tb = (in_a, tab_a) if p == 0 else (in_b, tab_b)
        sem_i, sem_o = (sem_ia, sem_oa) if p == 0 else (sem_ib, sem_ob)
        in_pend[p].wait()
        if out_pend[p] is not None:
            out_pend[p].wait()

        @plsc.parallel_loop(0, GROUPS, unroll=4)
        def _(g, vv=vv, tb=tb):
            sl = pl.ds(g * L, L)
            rep = rep_v[sl] != 0
            cells = idx_v[sl]
            sval = vv[sl] * plsc.load_gather(rec, [cells])
            plsc.store_scatter(tb, [cells], sval, mask=rep)

        def add_k(k, carry2, vv=vv, tb=tb):
            valid = (k * L + lanes) < n_nf
            pts = nf_pt_v[pl.ds(k * L, L)]
            pts = jnp.where(valid, pts, 0)
            cells = plsc.load_gather(idx_v, [pts])
            vals = plsc.load_gather(vv, [pts])
            sval = vals * plsc.load_gather(rec, [cells])

            def cond(a):
                return jnp.any(a)

            def body(a):
                plsc.store_scatter(claim, [cells], lanes, mask=a)
                got = plsc.load_gather(claim, [cells])
                win = a & (got == lanes)
                cur = plsc.load_gather(tb, [cells])
                plsc.store_scatter(tb, [cells], cur + sval, mask=win)
                return a & jnp.logical_not(win)

            lax.while_loop(cond, body, valid)
            return carry2
        lax.fori_loop(0, n_nf_vregs, add_k, 0)

        if ci + 2 < _CH_PER_W:
            in_pend[p] = pltpu.async_copy(c_hbm.at[b, c0 + ci + 2], vv, sem_i)
        out_pend[p] = pltpu.async_copy(tb, out_hbm.at[b, c0 + ci], sem_o)
    out_pend[0].wait()
    out_pend[1].wait()


def kernel(p, fc_pos_W, fc_pos_b, W0, b0, W1, b1, Ws, fc_c_W, fc_c_b):
    pt = jnp.transpose(p, (0, 2, 1))
    fwT = jnp.transpose(fc_pos_W)
    fbT = fc_pos_b[:, None]
    w0T = jnp.transpose(W0, (0, 2, 1))
    b0T = b0[:, :, None]
    w1T = jnp.transpose(W1, (0, 2, 1))
    b1T = b1[:, :, None]
    wsT = jnp.transpose(Ws, (0, 2, 1))
    fcwT = jnp.transpose(fc_c_W)
    fcbT = fc_c_b[:, None]

    idx, net = _prologue_call(pt, fwT, fbT, w0T[0], b0T[0], w1T[0], b1T[0],
                              wsT[0])
    for i in range(1, NB):
        pooled = _pool_call(idx, net)
        if i < NB - 1:
            net = _res_call(net, pooled, w0T[i], b0T[i], w1T[i], b1T[i],
                            wsT[i])
        else:
            c = _res_final_call(net, pooled, w0T[i], b0T[i], w1T[i], b1T[i],
                                wsT[i], fcwT, fcbT)
    plane = _mean_call(idx, c)
    return plane.reshape(B, C_DIM, RESO, RESO)
